# baseline (speedup 1.0000x reference)
import functools

import jax
import jax.numpy as jnp
from jax import lax
from jax.experimental import pallas as pl
from jax.experimental.pallas import tpu as pltpu
from jax.experimental.pallas import tpu_sc as plsc

_BATCH = 4
_SEQ = 4096
_D = 1024

_NC = 2
_NS = 16
_NW = _NC * _NS

_SROWS = _SEQ // _NW
_CH = 4
_NCHUNK = _SROWS // _CH
_NG = _D // 16


def _sc_body(x_hbm, p_hbm, o_hbm,
             posbuf, in0, in1, out0, out1,
             isem0, isem1, osem0, osem1):
    wid = lax.axis_index("c") * _NS + lax.axis_index("s")
    base = wid * _SROWS

    ins = (in0, in1)
    outs = (out0, out1)
    isems = (isem0, isem1)
    osems = (osem0, osem1)

    def chunk(j, carry):
        row0 = base + j * _CH
        pltpu.async_copy(x_hbm.at[pl.ds(row0, _CH)], ins[0], isems[0])
        pltpu.sync_copy(p_hbm.at[pl.ds(row0, _CH)], posbuf)
        for b in range(_BATCH):
            slot = b % 2
            if b < _BATCH - 1:
                pltpu.async_copy(
                    x_hbm.at[pl.ds((b + 1) * _SEQ + row0, _CH)],
                    ins[1 - slot], isems[1 - slot])
            pltpu.make_async_copy(
                x_hbm.at[pl.ds(0, _CH)], ins[slot], isems[slot]).wait()
            if b >= 2:
                pltpu.make_async_copy(
                    outs[slot], o_hbm.at[pl.ds(0, _CH)], osems[slot]).wait()
            else:
                @pl.when(j > 0)
                def _():
                    pltpu.make_async_copy(
                        outs[slot], o_hbm.at[pl.ds(0, _CH)],
                        osems[slot]).wait()

            def add_grp(g, c):
                sl = pl.ds(g * 16, 16)
                for r in range(_CH):
                    outs[slot][r, sl] = ins[slot][r, sl] + posbuf[r, sl]
                return c

            lax.fori_loop(0, _NG, add_grp, 0)
            pltpu.async_copy(
                outs[slot], o_hbm.at[pl.ds(b * _SEQ + row0, _CH)],
                osems[slot])
        return carry

    lax.fori_loop(0, _NCHUNK, chunk, 0)
    pltpu.make_async_copy(out0, o_hbm.at[pl.ds(0, _CH)], osem0).wait()
    pltpu.make_async_copy(out1, o_hbm.at[pl.ds(0, _CH)], osem1).wait()


_sc_add = functools.partial(
    pl.kernel,
    mesh=plsc.VectorSubcoreMesh(core_axis_name="c", subcore_axis_name="s"),
    out_type=jax.ShapeDtypeStruct((_BATCH * _SEQ, _D), jnp.float32),
    scratch_types=[
        pltpu.VMEM((_CH, _D), jnp.float32),
        pltpu.VMEM((_CH, _D), jnp.float32),
        pltpu.VMEM((_CH, _D), jnp.float32),
        pltpu.VMEM((_CH, _D), jnp.float32),
        pltpu.VMEM((_CH, _D), jnp.float32),
        pltpu.SemaphoreType.DMA,
        pltpu.SemaphoreType.DMA,
        pltpu.SemaphoreType.DMA,
        pltpu.SemaphoreType.DMA,
    ],
)(_sc_body)


def kernel(inputs, pos_table):
    out = _sc_add(inputs.reshape(_BATCH * _SEQ, _D), pos_table)
    return out.reshape(inputs.shape)


# speedup vs baseline: 1.6608x; 1.6608x over previous
import functools

import jax
import jax.numpy as jnp
from jax import lax
from jax.experimental import pallas as pl
from jax.experimental.pallas import tpu as pltpu
from jax.experimental.pallas import tpu_sc as plsc

_BATCH = 4
_SEQ = 4096
_D = 1024

_NC = 2
_NS = 16
_NW = _NC * _NS

_SROWS = _SEQ // _NW
_CH = 2
_NCHUNK = _SROWS // _CH
_NG = _D // 16


def _sc_body(x_hbm, p_hbm, o_hbm,
             i00, i01, i02, i03, i10, i11, i12, i13,
             o00, o01, o02, o03, o10, o11, o12, o13,
             pos0, pos1,
             isem0, isem1, psem0, psem1, osem0, osem1):
    wid = lax.axis_index("c") * _NS + lax.axis_index("s")
    base = wid * _SROWS

    ins = ((i00, i01, i02, i03), (i10, i11, i12, i13))
    outs = ((o00, o01, o02, o03), (o10, o11, o12, o13))
    poss = (pos0, pos1)
    isems = (isem0, isem1)
    psems = (psem0, psem1)
    osems = (osem0, osem1)

    for b in range(_BATCH):
        pltpu.async_copy(
            x_hbm.at[pl.ds(b * _SEQ + base, _CH)], ins[0][b], isem0)
    pltpu.async_copy(p_hbm.at[pl.ds(base, _CH)], pos0, psem0)

    def cpair(jc, carry):
        for jj in range(2):
            j = jc * 2 + jj
            slot = jj
            row0 = base + j * _CH
            nrow0 = row0 + _CH

            @pl.when(j + 1 < _NCHUNK)
            def _():
                for b in range(_BATCH):
                    pltpu.async_copy(
                        x_hbm.at[pl.ds(b * _SEQ + nrow0, _CH)],
                        ins[1 - slot][b], isems[1 - slot])
                pltpu.async_copy(
                    p_hbm.at[pl.ds(nrow0, _CH)], poss[1 - slot],
                    psems[1 - slot])

            for _k in range(_BATCH):
                pltpu.make_async_copy(
                    x_hbm.at[pl.ds(0, _CH)], ins[slot][0],
                    isems[slot]).wait()
            pltpu.make_async_copy(
                p_hbm.at[pl.ds(0, _CH)], poss[slot], psems[slot]).wait()

            @pl.when(j >= 2)
            def _():
                for _k in range(_BATCH):
                    pltpu.make_async_copy(
                        outs[slot][0], o_hbm.at[pl.ds(0, _CH)],
                        osems[slot]).wait()

            def add_grp(g, c):
                sl = pl.ds(g * 16, 16)
                for r in range(_CH):
                    pv = poss[slot][r, sl]
                    for b in range(_BATCH):
                        outs[slot][b][r, sl] = ins[slot][b][r, sl] + pv
                return c

            lax.fori_loop(0, _NG, add_grp, 0)

            for b in range(_BATCH):
                pltpu.async_copy(
                    outs[slot][b], o_hbm.at[pl.ds(b * _SEQ + row0, _CH)],
                    osems[slot])
        return carry

    lax.fori_loop(0, _NCHUNK // 2, cpair, 0)

    for s in range(2):
        for _k in range(_BATCH):
            pltpu.make_async_copy(
                outs[s][0], o_hbm.at[pl.ds(0, _CH)], osems[s]).wait()


_sc_add = functools.partial(
    pl.kernel,
    mesh=plsc.VectorSubcoreMesh(core_axis_name="c", subcore_axis_name="s"),
    out_type=jax.ShapeDtypeStruct((_BATCH * _SEQ, _D), jnp.float32),
    scratch_types=(
        [pltpu.VMEM((_CH, _D), jnp.float32) for _ in range(18)]
        + [pltpu.SemaphoreType.DMA for _ in range(6)]
    ),
)(_sc_body)


def kernel(inputs, pos_table):
    out = _sc_add(inputs.reshape(_BATCH * _SEQ, _D), pos_table)
    return out.reshape(inputs.shape)
